# baseline (speedup 1.0000x reference)
import functools

import jax
import jax.numpy as jnp
from jax import lax
from jax.experimental import pallas as pl
from jax.experimental.pallas import tpu as pltpu
from jax.experimental.pallas import tpu_sc as plsc

E = 8
D_IN = 1024
D_H = 2048
N = 8192

T = 256
NT = N // T + E
P = NT * T

NC = 2
NS = 16
NW = NC * NS
GCHUNK = 40


SCH = 32
SNCH = (N // NW) // SCH


def _row_scatter_body(aev_hbm, idx_hbm, out_hbm, idx_v, rows_a, rows_b,
                     gsem_a, gsem_b, osem_a, osem_b):
    wid = lax.axis_index("s") * NC + lax.axis_index("c")
    bpw = N // NW
    base = wid * bpw
    pltpu.sync_copy(idx_hbm.at[wid], idx_v)
    bufs, gsems, osems = (rows_a, rows_b), (gsem_a, gsem_b), (osem_a, osem_b)
    g = [None] * SNCH
    o = [None] * SNCH
    g[0] = pltpu.async_copy(
        aev_hbm.at[pl.ds(base, SCH)], bufs[0], gsems[0])
    for j in range(SNCH):
        b = j % 2
        g[j].wait()
        if j + 1 < SNCH:
            if j >= 1:
                o[j - 1].wait()
            g[j + 1] = pltpu.async_copy(
                aev_hbm.at[pl.ds(base + (j + 1) * SCH, SCH)],
                bufs[1 - b], gsems[1 - b])
        o[j] = pltpu.async_copy(
            bufs[b], out_hbm.at[idx_v.at[j]], osems[b])
    o[SNCH - 2].wait()
    o[SNCH - 1].wait()


def _sc_scatter_rows(aev, pos3):
    mesh = plsc.VectorSubcoreMesh(core_axis_name="c", subcore_axis_name="s")
    k = pl.kernel(
        _row_scatter_body,
        out_type=jax.ShapeDtypeStruct((P, D_IN), jnp.float32),
        mesh=mesh,
        scratch_types=[
            pltpu.VMEM((SNCH, SCH), jnp.int32),
            pltpu.VMEM((SCH, D_IN), jnp.float32),
            pltpu.VMEM((SCH, D_IN), jnp.float32),
            pltpu.SemaphoreType.DMA,
            pltpu.SemaphoreType.DMA,
            pltpu.SemaphoreType.DMA,
            pltpu.SemaphoreType.DMA,
        ],
    )
    return k(aev, pos3)


def _scalar_gather_body(vals_hbm, idx_hbm, out_hbm, vals_v, idx_v, out_v):
    wid = lax.axis_index("s") * NC + lax.axis_index("c")
    bpw = N // NW
    base = wid * bpw
    pltpu.sync_copy(vals_hbm, vals_v)
    pltpu.sync_copy(idx_hbm.at[pl.ds(base, bpw)], idx_v)

    def body(j, carry):
        idx16 = idx_v[pl.ds(j * 16, 16)]
        out_v[pl.ds(j * 16, 16)] = plsc.load_gather(vals_v, [idx16])
        return carry

    lax.fori_loop(0, bpw // 16, body, 0)
    pltpu.sync_copy(out_v, out_hbm.at[pl.ds(base, bpw)])


def _sc_gather_scalars(vals, idx):
    mesh = plsc.VectorSubcoreMesh(core_axis_name="c", subcore_axis_name="s")
    k = pl.kernel(
        _scalar_gather_body,
        out_type=jax.ShapeDtypeStruct((N,), jnp.float32),
        mesh=mesh,
        scratch_types=[
            pltpu.VMEM((P,), jnp.float32),
            pltpu.VMEM((N // NW,), jnp.int32),
            pltpu.VMEM((N // NW,), jnp.float32),
        ],
        compiler_params=pltpu.CompilerParams(needs_layout_passes=False),
    )
    return k(vals, idx)


def _celu(x):
    return jnp.where(x > 0, x, jnp.exp(x) - 1.0)


def _mlp_body(ctrl_ref, x_ref, w0_hbm, w1_hbm, b0_ref, b1_ref, w2_ref, sv_ref,
              out_ref, w0_buf, w1_buf, sems):
    i = pl.program_id(0)
    e = ctrl_ref[0, i]
    first = ctrl_ref[1, i]
    ldexp = ctrl_ref[2, i]
    slot = ctrl_ref[3, i]
    nact = ctrl_ref[0, NT]

    def w_copy(expert, s):
        return (
            pltpu.make_async_copy(w0_hbm.at[expert], w0_buf.at[s],
                                  sems.at[s, 0]),
            pltpu.make_async_copy(w1_hbm.at[expert], w1_buf.at[s],
                                  sems.at[s, 1]),
        )

    @pl.when(i == 0)
    def _():
        for c in w_copy(e, 0):
            c.start()

    @pl.when((first == 1) & (ldexp >= 0))
    def _():
        @pl.when(slot == 0)
        def _():
            for c in w_copy(ldexp, 1):
                c.start()

        @pl.when(slot == 1)
        def _():
            for c in w_copy(ldexp, 0):
                c.start()

    @pl.when((first == 1) & (i >= nact))
    def _():
        @pl.when(slot == 0)
        def _():
            for c in w_copy(e, 0):
                c.wait()

        @pl.when(slot == 1)
        def _():
            for c in w_copy(e, 1):
                c.wait()

    def compute(s, is_first):
        c0, c1 = w_copy(e, s)
        if is_first:
            c0.wait()
        h = jnp.dot(x_ref[...], w0_buf[s], preferred_element_type=jnp.float32)
        h = _celu(h + b0_ref[0])
        if is_first:
            c1.wait()
        h = jnp.dot(h, w1_buf[s], preferred_element_type=jnp.float32)
        h = _celu(h + b1_ref[0])
        y = jnp.sum(h * w2_ref[0], axis=1) + sv_ref[e]
        out_ref[0, 0, :] = y

    @pl.when(i < nact)
    def _():
        for s in (0, 1):
            for f in (0, 1):
                @pl.when((slot == s) & (first == f))
                def _(s=s, f=f):
                    compute(s, bool(f))


def _tc_mlp(ctrl, x_sorted, W0, b0r, W1, b1r, w2r, svec):
    grid_spec = pltpu.PrefetchScalarGridSpec(
        num_scalar_prefetch=1,
        grid=(NT,),
        in_specs=[
            pl.BlockSpec((T, D_IN), lambda i, ct: (i, 0)),
            pl.BlockSpec(memory_space=pl.ANY),
            pl.BlockSpec(memory_space=pl.ANY),
            pl.BlockSpec((1, 1, D_H), lambda i, ct: (ct[0, i], 0, 0)),
            pl.BlockSpec((1, 1, D_H), lambda i, ct: (ct[0, i], 0, 0)),
            pl.BlockSpec((1, 1, D_H), lambda i, ct: (ct[0, i], 0, 0)),
            pl.BlockSpec(memory_space=pltpu.MemorySpace.SMEM),
        ],
        out_specs=pl.BlockSpec((1, 1, T), lambda i, ct: (i, 0, 0)),
        scratch_shapes=[
            pltpu.VMEM((2, D_IN, D_H), jnp.float32),
            pltpu.VMEM((2, D_H, D_H), jnp.float32),
            pltpu.SemaphoreType.DMA((2, 2)),
        ],
    )
    return pl.pallas_call(
        _mlp_body,
        grid_spec=grid_spec,
        out_shape=jax.ShapeDtypeStruct((NT, 1, T), jnp.float32),
        compiler_params=pltpu.CompilerParams(
            dimension_semantics=("arbitrary",),
            vmem_limit_bytes=100 * 1024 * 1024,
        ),
    )(ctrl, x_sorted, W0, W1, b0r, b1r, w2r, svec)


def _routing_plan(index):
    onehot = index[:, None] == jnp.arange(E, dtype=jnp.int32)[None, :]
    counts = jnp.sum(onehot.astype(jnp.int32), axis=0)
    tiles_e = (counts + T - 1) // T
    pad_e = tiles_e * T
    off = jnp.concatenate(
        [jnp.zeros((1,), jnp.int32), jnp.cumsum(pad_e)[:-1].astype(jnp.int32)])
    ranks = jnp.cumsum(onehot.astype(jnp.int32), axis=0) - 1
    r = jnp.sum(jnp.where(onehot, ranks, 0), axis=1)
    tok_off = jnp.sum(jnp.where(onehot, off[None, :], 0), axis=1)
    pos = tok_off + r
    tile_cum = jnp.cumsum(tiles_e)
    tmap = jnp.sum(
        (jnp.arange(NT, dtype=jnp.int32)[:, None] >= tile_cum[None, :])
        .astype(jnp.int32), axis=1)
    tmap = jnp.minimum(tmap, E - 1).astype(jnp.int32)
    n_active = tile_cum[-1].astype(jnp.int32)

    first = jnp.concatenate(
        [jnp.ones((1,), jnp.int32),
         (tmap[1:] != tmap[:-1]).astype(jnp.int32)])
    g = jnp.cumsum(first) - 1
    slot = (g % 2).astype(jnp.int32)
    gexp = jnp.full((NT + 1,), -1, jnp.int32).at[g].set(tmap)
    ldexp = jnp.where(first == 1, gexp[jnp.minimum(g + 1, NT)], -1)

    ctrl = jnp.stack([
        jnp.concatenate([tmap, n_active[None]]),
        jnp.concatenate([first, jnp.zeros((1,), jnp.int32)]),
        jnp.concatenate([ldexp, jnp.full((1,), -1, jnp.int32)]),
        jnp.concatenate([slot, jnp.zeros((1,), jnp.int32)]),
    ])
    return ctrl, pos


def kernel(index, aev, W0, b0, W1, b1, W2, b2):
    index = index.astype(jnp.int32)
    ctrl, pos = _routing_plan(index)
    svec = b2[:, 0] + 0.1 * jnp.arange(E, dtype=jnp.float32)
    b0r = b0.reshape(E, 1, D_H)
    b1r = b1.reshape(E, 1, D_H)
    w2r = W2[:, :, 0].reshape(E, 1, D_H)
    x_sorted = _sc_scatter_rows(aev, pos.reshape(NW, SNCH, SCH))
    y = _tc_mlp(ctrl, x_sorted, W0, b0r, W1, b1r, w2r, svec)
    return _sc_gather_scalars(y.reshape(P), pos)


# speedup vs baseline: 1.0084x; 1.0084x over previous
import functools

import jax
import jax.numpy as jnp
from jax import lax
from jax.experimental import pallas as pl
from jax.experimental.pallas import tpu as pltpu
from jax.experimental.pallas import tpu_sc as plsc

E = 8
D_IN = 1024
D_H = 2048
N = 8192

T = 256
NT = N // T + E
P = NT * T

NC = 2
NS = 16
NW = NC * NS
GCHUNK = 40


SCH = 32
SNCH = (N // NW) // SCH


def _row_scatter_body(aev_hbm, idx_hbm, out_hbm, idx_v, rows_a, rows_b,
                     gsem_a, gsem_b, osem_a, osem_b):
    wid = lax.axis_index("s") * NC + lax.axis_index("c")
    bpw = N // NW
    base = wid * bpw
    pltpu.sync_copy(idx_hbm.at[wid], idx_v)
    bufs, gsems, osems = (rows_a, rows_b), (gsem_a, gsem_b), (osem_a, osem_b)
    g = [None] * SNCH
    o = [None] * SNCH
    g[0] = pltpu.async_copy(
        aev_hbm.at[pl.ds(base, SCH)], bufs[0], gsems[0])
    for j in range(SNCH):
        b = j % 2
        g[j].wait()
        if j + 1 < SNCH:
            if j >= 1:
                o[j - 1].wait()
            g[j + 1] = pltpu.async_copy(
                aev_hbm.at[pl.ds(base + (j + 1) * SCH, SCH)],
                bufs[1 - b], gsems[1 - b])
        o[j] = pltpu.async_copy(
            bufs[b], out_hbm.at[idx_v.at[j]], osems[b])
    o[SNCH - 2].wait()
    o[SNCH - 1].wait()


def _sc_scatter_rows(aev, pos3):
    mesh = plsc.VectorSubcoreMesh(core_axis_name="c", subcore_axis_name="s")
    k = pl.kernel(
        _row_scatter_body,
        out_type=jax.ShapeDtypeStruct((P, D_IN), jnp.float32),
        mesh=mesh,
        scratch_types=[
            pltpu.VMEM((SNCH, SCH), jnp.int32),
            pltpu.VMEM((SCH, D_IN), jnp.float32),
            pltpu.VMEM((SCH, D_IN), jnp.float32),
            pltpu.SemaphoreType.DMA,
            pltpu.SemaphoreType.DMA,
            pltpu.SemaphoreType.DMA,
            pltpu.SemaphoreType.DMA,
        ],
    )
    return k(aev, pos3)


def _scalar_gather_body(vals_hbm, idx_hbm, out_hbm, vals_v, idx_v, out_v):
    wid = lax.axis_index("s") * NC + lax.axis_index("c")
    bpw = N // NW
    base = wid * bpw
    pltpu.sync_copy(vals_hbm, vals_v)
    pltpu.sync_copy(idx_hbm.at[pl.ds(base, bpw)], idx_v)

    def body(j, carry):
        idx16 = idx_v[pl.ds(j * 16, 16)]
        out_v[pl.ds(j * 16, 16)] = plsc.load_gather(vals_v, [idx16])
        return carry

    lax.fori_loop(0, bpw // 16, body, 0)
    pltpu.sync_copy(out_v, out_hbm.at[pl.ds(base, bpw)])


def _sc_gather_scalars(vals, idx):
    mesh = plsc.VectorSubcoreMesh(core_axis_name="c", subcore_axis_name="s")
    k = pl.kernel(
        _scalar_gather_body,
        out_type=jax.ShapeDtypeStruct((N,), jnp.float32),
        mesh=mesh,
        scratch_types=[
            pltpu.VMEM((P,), jnp.float32),
            pltpu.VMEM((N // NW,), jnp.int32),
            pltpu.VMEM((N // NW,), jnp.float32),
        ],
        compiler_params=pltpu.CompilerParams(needs_layout_passes=False),
    )
    return k(vals, idx)


def _celu(x):
    return jnp.where(x > 0, x, jnp.exp(x) - 1.0)


def _mlp_body(ctrl_ref, x_ref, w0_hbm, w1_hbm, b0_ref, b1_ref, w2_ref, sv_ref,
              out_ref, w0_buf, w1_buf, sems):
    i = pl.program_id(0)
    e = ctrl_ref[0, i]
    first = ctrl_ref[1, i]
    ldexp = ctrl_ref[2, i]
    slot = ctrl_ref[3, i]
    nact = ctrl_ref[0, NT]

    def w_copy(expert, s):
        return (
            pltpu.make_async_copy(w0_hbm.at[expert], w0_buf.at[s],
                                  sems.at[s, 0]),
            pltpu.make_async_copy(w1_hbm.at[expert, pl.ds(0, D_H // 2)],
                                  w1_buf.at[s, pl.ds(0, D_H // 2)],
                                  sems.at[s, 1]),
            pltpu.make_async_copy(w1_hbm.at[expert, pl.ds(D_H // 2, D_H // 2)],
                                  w1_buf.at[s, pl.ds(D_H // 2, D_H // 2)],
                                  sems.at[s, 2]),
        )

    @pl.when(i == 0)
    def _():
        for c in w_copy(e, 0):
            c.start()

    @pl.when((first == 1) & (ldexp >= 0))
    def _():
        @pl.when(slot == 0)
        def _():
            for c in w_copy(ldexp, 1):
                c.start()

        @pl.when(slot == 1)
        def _():
            for c in w_copy(ldexp, 0):
                c.start()

    @pl.when(first == 1)
    def _():
        @pl.when(slot == 0)
        def _():
            for c in w_copy(e, 0):
                c.wait()

        @pl.when(slot == 1)
        def _():
            for c in w_copy(e, 1):
                c.wait()

    def compute(s):
        h = jnp.dot(x_ref[...], w0_buf[s], preferred_element_type=jnp.float32)
        h = _celu(h + b0_ref[0])
        h = jnp.dot(h, w1_buf[s], preferred_element_type=jnp.float32)
        h = _celu(h + b1_ref[0])
        y = jnp.sum(h * w2_ref[0], axis=1) + sv_ref[e]
        out_ref[0, 0, :] = y

    @pl.when(i < nact)
    def _():
        @pl.when(slot == 0)
        def _():
            compute(0)

        @pl.when(slot == 1)
        def _():
            compute(1)


def _tc_mlp(ctrl, x_sorted, W0, b0r, W1, b1r, w2r, svec):
    grid_spec = pltpu.PrefetchScalarGridSpec(
        num_scalar_prefetch=1,
        grid=(NT,),
        in_specs=[
            pl.BlockSpec((T, D_IN), lambda i, ct: (i, 0)),
            pl.BlockSpec(memory_space=pl.ANY),
            pl.BlockSpec(memory_space=pl.ANY),
            pl.BlockSpec((1, 1, D_H), lambda i, ct: (ct[0, i], 0, 0)),
            pl.BlockSpec((1, 1, D_H), lambda i, ct: (ct[0, i], 0, 0)),
            pl.BlockSpec((1, 1, D_H), lambda i, ct: (ct[0, i], 0, 0)),
            pl.BlockSpec(memory_space=pltpu.MemorySpace.SMEM),
        ],
        out_specs=pl.BlockSpec((1, 1, T), lambda i, ct: (i, 0, 0)),
        scratch_shapes=[
            pltpu.VMEM((2, D_IN, D_H), jnp.float32),
            pltpu.VMEM((2, D_H, D_H), jnp.float32),
            pltpu.SemaphoreType.DMA((2, 3)),
        ],
    )
    return pl.pallas_call(
        _mlp_body,
        grid_spec=grid_spec,
        out_shape=jax.ShapeDtypeStruct((NT, 1, T), jnp.float32),
        compiler_params=pltpu.CompilerParams(
            dimension_semantics=("arbitrary",),
            vmem_limit_bytes=100 * 1024 * 1024,
        ),
    )(ctrl, x_sorted, W0, W1, b0r, b1r, w2r, svec)


def _routing_plan(index):
    onehot = index[:, None] == jnp.arange(E, dtype=jnp.int32)[None, :]
    counts = jnp.sum(onehot.astype(jnp.int32), axis=0)
    tiles_e = (counts + T - 1) // T
    pad_e = tiles_e * T
    off = jnp.concatenate(
        [jnp.zeros((1,), jnp.int32), jnp.cumsum(pad_e)[:-1].astype(jnp.int32)])
    ranks = jnp.cumsum(onehot.astype(jnp.int32), axis=0) - 1
    r = jnp.sum(jnp.where(onehot, ranks, 0), axis=1)
    tok_off = jnp.sum(jnp.where(onehot, off[None, :], 0), axis=1)
    pos = tok_off + r
    tile_cum = jnp.cumsum(tiles_e)
    tmap = jnp.sum(
        (jnp.arange(NT, dtype=jnp.int32)[:, None] >= tile_cum[None, :])
        .astype(jnp.int32), axis=1)
    tmap = jnp.minimum(tmap, E - 1).astype(jnp.int32)
    n_active = tile_cum[-1].astype(jnp.int32)

    first = jnp.concatenate(
        [jnp.ones((1,), jnp.int32),
         (tmap[1:] != tmap[:-1]).astype(jnp.int32)])
    g = jnp.cumsum(first) - 1
    slot = (g % 2).astype(jnp.int32)
    gexp = jnp.full((NT + 1,), -1, jnp.int32).at[g].set(tmap)
    ldexp = jnp.where(first == 1, gexp[jnp.minimum(g + 1, NT)], -1)

    ctrl = jnp.stack([
        jnp.concatenate([tmap, n_active[None]]),
        jnp.concatenate([first, jnp.zeros((1,), jnp.int32)]),
        jnp.concatenate([ldexp, jnp.full((1,), -1, jnp.int32)]),
        jnp.concatenate([slot, jnp.zeros((1,), jnp.int32)]),
    ])
    return ctrl, pos


def kernel(index, aev, W0, b0, W1, b1, W2, b2):
    index = index.astype(jnp.int32)
    ctrl, pos = _routing_plan(index)
    svec = b2[:, 0] + 0.1 * jnp.arange(E, dtype=jnp.float32)
    b0r = b0.reshape(E, 1, D_H)
    b1r = b1.reshape(E, 1, D_H)
    w2r = W2[:, :, 0].reshape(E, 1, D_H)
    x_sorted = _sc_scatter_rows(aev, pos.reshape(NW, SNCH, SCH))
    y = _tc_mlp(ctrl, x_sorted, W0, b0r, W1, b1r, w2r, svec)
    return _sc_gather_scalars(y.reshape(P), pos)
